# Initial kernel scaffold; baseline (speedup 1.0000x reference)
import jax
import jax.numpy as jnp
from jax.experimental import pallas as pl


def kernel(user_emb, item_emb, mat_indices, mat_values):
    raise NotImplementedError("write your pallas kernel here")


# speedup vs baseline: 4.8352x; 4.8352x over previous
import functools

import jax
import jax.numpy as jnp
from jax import lax
from jax.experimental import pallas as pl
from jax.experimental.pallas import tpu as pltpu
from jax.experimental.pallas import tpu_sc as plsc

D = 128
LANES = 16
NS = 16
K = 128


def _make_agg(n_users, n_items, e_pad):
    edges_per_tile = e_pad // NS
    chunks = edges_per_tile // K
    rows_main = 624
    rows_last = n_users - (NS - 1) * rows_main

    mesh = plsc.VectorSubcoreMesh(core_axis_name="c", subcore_axis_name="s",
                                  num_cores=2, num_subcores=NS)

    @functools.partial(
        pl.kernel,
        out_type=(
            jax.ShapeDtypeStruct((n_users, D), jnp.float32),
            jax.ShapeDtypeStruct((n_items, D), jnp.float32),
        ),
        mesh=mesh,
        scratch_types=[
            pltpu.VMEM_SHARED((n_users, D), jnp.float32),
            pltpu.VMEM((K,), jnp.int32),
            pltpu.VMEM((K,), jnp.int32),
            pltpu.VMEM((K,), jnp.float32),
            pltpu.VMEM((K, D), jnp.float32),
            pltpu.SemaphoreType.DMA,
        ],
    )
    def agg(user_emb, item_emb, row_idx, col_idx, vals,
            out_u, out_i, acc, idxs_v, idxd_v, vals_v, rows_v, sem):
        cid = lax.axis_index("c")
        sid = lax.axis_index("s")

        zeros16 = jnp.zeros((LANES,), jnp.float32)

        def zero_row(r, carry):
            for j in range(D // LANES):
                rows_v[r, pl.ds(j * LANES, LANES)] = zeros16
            return carry

        lax.fori_loop(0, K, zero_row, 0)

        @pl.when(sid < NS - 1)
        def _():
            base_r = sid * rows_main
            for i in range(6):
                pltpu.sync_copy(rows_v.at[pl.ds(0, 104)],
                                acc.at[pl.ds(base_r + i * 104, 104)])

        @pl.when(sid == NS - 1)
        def _():
            base_r = (NS - 1) * rows_main
            for i in range(rows_last // K):
                pltpu.sync_copy(rows_v.at[pl.ds(0, K)],
                                acc.at[pl.ds(base_r + i * K, K)])

        plsc.subcore_barrier()

        def run(dst_hbm, src_hbm, emb_hbm, out_hbm):
            def chunk_body(c, carry):
                base = sid * edges_per_tile + c * K
                pltpu.sync_copy(src_hbm.at[pl.ds(base, K)], idxs_v)
                pltpu.sync_copy(dst_hbm.at[pl.ds(base, K)], idxd_v)
                pltpu.sync_copy(vals.at[pl.ds(base, K)], vals_v)
                pltpu.async_copy(emb_hbm.at[idxs_v], rows_v, sem).wait()

                def scale_grp(g, c2):
                    vals16 = vals_v[pl.ds(g * LANES, LANES)]
                    for e in range(LANES):
                        v = vals16[e]
                        r = g * LANES + e
                        for j in range(D // LANES):
                            sl = pl.ds(j * LANES, LANES)
                            rows_v[r, sl] = rows_v[r, sl] * v
                    return c2

                lax.fori_loop(0, K // LANES, scale_grp, 0)
                pltpu.sync_copy(rows_v, acc.at[idxd_v], add=True)
                return carry

            lax.fori_loop(0, chunks, chunk_body, 0)
            plsc.subcore_barrier()

            @pl.when(sid < NS - 1)
            def _():
                rb = sid * rows_main
                pltpu.sync_copy(acc.at[pl.ds(rb, rows_main)],
                                out_hbm.at[pl.ds(rb, rows_main)])

            @pl.when(sid == NS - 1)
            def _():
                rb = (NS - 1) * rows_main
                pltpu.sync_copy(acc.at[pl.ds(rb, rows_last)],
                                out_hbm.at[pl.ds(rb, rows_last)])

        @pl.when(cid == 0)
        def _():
            run(row_idx, col_idx, item_emb, out_u)

        @pl.when(cid == 1)
        def _():
            run(col_idx, row_idx, user_emb, out_i)

    return agg


def kernel(user_emb, item_emb, mat_indices, mat_values):
    n_users = user_emb.shape[0]
    n_items = item_emb.shape[0]
    e = mat_values.shape[0]
    e_pad = ((e + NS * K - 1) // (NS * K)) * (NS * K)
    pad = e_pad - e
    row = mat_indices[0]
    col = mat_indices[1]
    if pad:
        zi = jnp.zeros((pad,), jnp.int32)
        row = jnp.concatenate([row, zi])
        col = jnp.concatenate([col, zi])
        mat_values = jnp.concatenate([mat_values,
                                      jnp.zeros((pad,), jnp.float32)])
    agg = _make_agg(n_users, n_items, e_pad)
    return agg(user_emb, item_emb, row, col, mat_values)
